# baseline (speedup 1.0000x reference)
import functools

import jax
import jax.numpy as jnp
from jax import lax
from jax.experimental import pallas as pl
from jax.experimental.pallas import tpu as pltpu
from jax.experimental.pallas import tpu_sc as plsc

S, H, I, E, K = 2048, 768, 3072, 8, 2
T = 512
NT = (S * K) // T + E
P = NT * T

_NEG = -1e30


def _router_body(hs_ref, g_ref, b_ref, rw_ref, rb_ref,
                 h_ref, idx_ref, w_ref, loss_ref):
    x = hs_ref[...]
    mu = jnp.mean(x, axis=1, keepdims=True)
    var = jnp.mean((x - mu) ** 2, axis=1, keepdims=True)
    h = (x - mu) * jax.lax.rsqrt(var + 1e-5) * g_ref[...] + b_ref[...]
    h_ref[...] = h
    logits = jnp.dot(h, rw_ref[...], preferred_element_type=jnp.float32)
    logits = logits + rb_ref[...]
    m = jnp.max(logits, axis=1, keepdims=True)
    ex = jnp.exp(logits - m)
    probs = ex / jnp.sum(ex, axis=1, keepdims=True)
    u = jnp.sum(probs, axis=0, keepdims=True) * (1.0 / S)
    loss = E * jnp.sum(u * u) - 1.0
    loss_ref[...] = jnp.full((1, 1), 0.0, jnp.float32) + loss
    lane = jax.lax.broadcasted_iota(jnp.int32, (S, E), 1)
    m1 = jnp.max(logits, axis=1, keepdims=True)
    i1 = jnp.min(jnp.where(logits == m1, lane, E), axis=1, keepdims=True)
    neg = jnp.where(lane == i1, _NEG, logits)
    m2 = jnp.max(neg, axis=1, keepdims=True)
    i2 = jnp.min(jnp.where(neg == m2, lane, E), axis=1, keepdims=True)
    e2 = jnp.exp(m2 - m1)
    wa = 1.0 / (1.0 + e2)
    wb = e2 / (1.0 + e2)
    lane2 = jax.lax.broadcasted_iota(jnp.int32, (S, K), 1)
    idx_ref[...] = jnp.where(lane2 == 0, i1, i2)
    w_ref[...] = jnp.where(lane2 == 0, wa, wb)


def _router(hs, g, b, rw, rb):
    return pl.pallas_call(
        _router_body,
        out_shape=[
            jax.ShapeDtypeStruct((S, H), jnp.float32),
            jax.ShapeDtypeStruct((S, K), jnp.int32),
            jax.ShapeDtypeStruct((S, K), jnp.float32),
            jax.ShapeDtypeStruct((1, 1), jnp.float32),
        ],
    )(hs, g, b, rw, rb)


def _ffn_body(te_ref, xs_ref, w1_ref, b1_ref, w2_ref, b2_ref, ws_ref, ys_ref):
    t = pl.program_id(0)

    @pl.when(t < te_ref[NT])
    def _():
        x = xs_ref[...]
        a = jnp.dot(x, w1_ref[0], preferred_element_type=jnp.float32)
        a = a + b1_ref[0]
        a = a * 0.5 * (1.0 + jax.lax.erf(a * 0.7071067811865476))
        y = jnp.dot(a, w2_ref[0], preferred_element_type=jnp.float32)
        y = y + b2_ref[0]
        ys_ref[...] = y * ws_ref[0]


def _ffn(xs, w1, b1, w2, b2, ws3, te_tab):
    grid_spec = pltpu.PrefetchScalarGridSpec(
        num_scalar_prefetch=1,
        grid=(NT,),
        in_specs=[
            pl.BlockSpec((T, H), lambda t, te: (jnp.minimum(t, te[NT] - 1), 0)),
            pl.BlockSpec((1, H, I), lambda t, te: (te[t], 0, 0)),
            pl.BlockSpec((1, 1, I), lambda t, te: (te[t], 0, 0)),
            pl.BlockSpec((1, I, H), lambda t, te: (te[t], 0, 0)),
            pl.BlockSpec((1, 1, H), lambda t, te: (te[t], 0, 0)),
            pl.BlockSpec((1, T, 1),
                         lambda t, te: (jnp.minimum(t, te[NT] - 1), 0, 0)),
        ],
        out_specs=pl.BlockSpec(
            (T, H), lambda t, te: (jnp.minimum(t, te[NT] - 1), 0)),
    )
    return pl.pallas_call(
        _ffn_body,
        grid_spec=grid_spec,
        out_shape=jax.ShapeDtypeStruct((P, H), jnp.float32),
    )(te_tab, xs, w1, b1, w2, b2, ws3)


_SC_MESH = plsc.VectorSubcoreMesh(core_axis_name="c", subcore_axis_name="s")
_NW = 32
_GCH = 32
_CCH = 32


_TPW = S // _NW


def _sc_dispatch_body(h_hbm, dest2_hbm, wf2_hbm, xs_hbm, ws_hbm,
                      buf, idx_v, wv, lsem, sem):
    wid = lax.axis_index("s") * 2 + lax.axis_index("c")
    t0 = wid * _TPW
    pltpu.sync_copy(dest2_hbm.at[0, pl.ds(t0, _TPW)], idx_v.at[0])
    pltpu.sync_copy(dest2_hbm.at[1, pl.ds(t0, _TPW)], idx_v.at[1])
    wcp = [
        pltpu.async_copy(wf2_hbm.at[0, pl.ds(t0, _TPW)], wv.at[0], lsem),
        pltpu.async_copy(wf2_hbm.at[1, pl.ds(t0, _TPW)], wv.at[1], lsem),
        pltpu.async_copy(h_hbm.at[pl.ds(t0, _TPW)], buf, lsem),
    ]
    for c in wcp:
        c.wait()
    cps = [
        pltpu.async_copy(buf, xs_hbm.at[idx_v.at[0]], sem),
        pltpu.async_copy(buf, xs_hbm.at[idx_v.at[1]], sem),
        pltpu.async_copy(wv.at[0], ws_hbm.at[idx_v.at[0]], sem),
        pltpu.async_copy(wv.at[1], ws_hbm.at[idx_v.at[1]], sem),
    ]
    for c in cps:
        c.wait()


@functools.partial(
    pl.kernel, mesh=_SC_MESH,
    out_type=[
        jax.ShapeDtypeStruct((P, H), jnp.float32),
        jax.ShapeDtypeStruct((P,), jnp.float32),
    ],
    scratch_types=[
        pltpu.VMEM((_TPW, H), jnp.float32),
        pltpu.VMEM((K, _TPW), jnp.int32),
        pltpu.VMEM((K, _TPW), jnp.float32),
        pltpu.SemaphoreType.DMA,
        pltpu.SemaphoreType.DMA,
    ],
)
def _sc_dispatch(h_hbm, dest2_hbm, wf2_hbm, xs_hbm, ws_hbm,
                 buf, idx_v, wv, lsem, sem):
    _sc_dispatch_body(h_hbm, dest2_hbm, wf2_hbm, xs_hbm, ws_hbm,
                      buf, idx_v, wv, lsem, sem)


def _sc_combine_body(ys_hbm, dest_hbm, res_hbm, out_hbm,
                     idx_v, rows_v, acc_v, sem):
    wid = lax.axis_index("s") * 2 + lax.axis_index("c")
    for c in range(_TPW // _CCH):
        t0 = wid * _TPW + c * _CCH
        pltpu.sync_copy(dest_hbm.at[pl.ds(K * t0, K * _CCH)], idx_v)
        gat = pltpu.async_copy(ys_hbm.at[idx_v], rows_v, sem)
        pltpu.sync_copy(res_hbm.at[pl.ds(t0, _CCH)], acc_v)
        gat.wait()

        def _row(i, _):
            def _vec(j, _):
                sl = pl.ds(j * 16, 16)
                acc_v[i, sl] = (acc_v[i, sl] + rows_v[2 * i, sl]
                                + rows_v[2 * i + 1, sl])
                return 0
            return lax.fori_loop(0, H // 16, _vec, 0)

        lax.fori_loop(0, _CCH, _row, 0)
        pltpu.sync_copy(acc_v, out_hbm.at[pl.ds(t0, _CCH)])


@functools.partial(
    pl.kernel, mesh=_SC_MESH,
    out_type=jax.ShapeDtypeStruct((S, H), jnp.float32),
    scratch_types=[
        pltpu.VMEM((K * _CCH,), jnp.int32),
        pltpu.VMEM((K * _CCH, H), jnp.float32),
        pltpu.VMEM((_CCH, H), jnp.float32),
        pltpu.SemaphoreType.DMA,
    ],
)
def _sc_combine(ys_hbm, dest_hbm, res_hbm, out_hbm,
                idx_v, rows_v, acc_v, sem):
    _sc_combine_body(ys_hbm, dest_hbm, res_hbm, out_hbm,
                     idx_v, rows_v, acc_v, sem)


def _combine_body(res_ref, yu_ref, out_ref):
    out_ref[...] = res_ref[...] + yu_ref[:, 0, :] + yu_ref[:, 1, :]


def _combine(res, yu):
    TC = 256
    return pl.pallas_call(
        _combine_body,
        grid=(S // TC,),
        in_specs=[
            pl.BlockSpec((TC, H), lambda i: (i, 0)),
            pl.BlockSpec((TC, K, H), lambda i: (i, 0, 0)),
        ],
        out_specs=pl.BlockSpec((TC, H), lambda i: (i, 0)),
        out_shape=jax.ShapeDtypeStruct((S, H), jnp.float32),
    )(res, yu)


def kernel(hidden_states, ln_gamma, ln_beta, router_w, router_b,
           w1, b1, w2, b2, training):
    hs = hidden_states.reshape(S, H)
    g2 = ln_gamma.reshape(1, H)
    b2d = ln_beta.reshape(1, H)
    rb2 = router_b.reshape(1, E)
    b1r = b1.reshape(E, 1, I)
    b2r = b2.reshape(E, 1, H)

    h, idx2, wts2, loss = _router(hs, g2, b2d, router_w, rb2)

    ef = idx2.reshape(-1)
    wf = wts2.reshape(-1)
    onehot = jax.nn.one_hot(ef, E, dtype=jnp.float32)
    csum = jnp.cumsum(onehot, axis=0)
    rank = jnp.sum(csum * onehot, axis=1).astype(jnp.int32) - 1
    counts = csum[-1].astype(jnp.int32)
    tiles_e = (counts + (T - 1)) // T
    tcum = jnp.cumsum(tiles_e)
    pstart = T * (tcum - tiles_e)
    dest = (jnp.sum(onehot * pstart.astype(jnp.float32)[None, :],
                    axis=1).astype(jnp.int32) + rank)
    dest2 = dest.reshape(S, K).T
    wf2 = wf.reshape(S, K).T
    ntiles = tcum[-1]
    tile_id = jnp.arange(NT, dtype=jnp.int32)
    te_raw = jnp.searchsorted(tcum, tile_id, side='right').astype(jnp.int32)
    last_e = jnp.searchsorted(tcum, ntiles - 1, side='right').astype(jnp.int32)
    te = jnp.where(tile_id < ntiles, jnp.minimum(te_raw, E - 1), last_e)
    te_tab = jnp.concatenate([te, ntiles.astype(jnp.int32)[None]])

    xs, ws = _sc_dispatch(h, dest2, wf2)
    ws3 = ws.reshape(NT, T, 1)

    ys = _ffn(xs, w1, b1r, w2, b2r, ws3, te_tab)

    out = _sc_combine(ys, dest, hs)

    return (out.reshape(1, S, H), loss[0, 0])


# speedup vs baseline: 1.0832x; 1.0832x over previous
import functools

import jax
import jax.numpy as jnp
from jax import lax
from jax.experimental import pallas as pl
from jax.experimental.pallas import tpu as pltpu
from jax.experimental.pallas import tpu_sc as plsc

S, H, I, E, K = 2048, 768, 3072, 8, 2
T = 512
NT = (S * K) // T + E
P = NT * T

_NEG = -1e30


def _router_body(hs_ref, g_ref, b_ref, rw_ref, rb_ref,
                 h_ref, idx_ref, w_ref, loss_ref):
    x = hs_ref[...]
    mu = jnp.mean(x, axis=1, keepdims=True)
    var = jnp.mean((x - mu) ** 2, axis=1, keepdims=True)
    h = (x - mu) * jax.lax.rsqrt(var + 1e-5) * g_ref[...] + b_ref[...]
    h_ref[...] = h
    logits = jnp.dot(h, rw_ref[...], preferred_element_type=jnp.float32)
    logits = logits + rb_ref[...]
    m = jnp.max(logits, axis=1, keepdims=True)
    ex = jnp.exp(logits - m)
    probs = ex / jnp.sum(ex, axis=1, keepdims=True)
    u = jnp.sum(probs, axis=0, keepdims=True) * (1.0 / S)
    loss = E * jnp.sum(u * u) - 1.0
    loss_ref[...] = jnp.full((1, 1), 0.0, jnp.float32) + loss
    lane = jax.lax.broadcasted_iota(jnp.int32, (S, E), 1)
    m1 = jnp.max(logits, axis=1, keepdims=True)
    i1 = jnp.min(jnp.where(logits == m1, lane, E), axis=1, keepdims=True)
    neg = jnp.where(lane == i1, _NEG, logits)
    m2 = jnp.max(neg, axis=1, keepdims=True)
    i2 = jnp.min(jnp.where(neg == m2, lane, E), axis=1, keepdims=True)
    e2 = jnp.exp(m2 - m1)
    wa = 1.0 / (1.0 + e2)
    wb = e2 / (1.0 + e2)
    lane2 = jax.lax.broadcasted_iota(jnp.int32, (S, K), 1)
    idx_ref[...] = jnp.where(lane2 == 0, i1, i2)
    w_ref[...] = jnp.where(lane2 == 0, wa, wb)


def _router(hs, g, b, rw, rb):
    return pl.pallas_call(
        _router_body,
        out_shape=[
            jax.ShapeDtypeStruct((S, H), jnp.float32),
            jax.ShapeDtypeStruct((S, K), jnp.int32),
            jax.ShapeDtypeStruct((S, K), jnp.float32),
            jax.ShapeDtypeStruct((1, 1), jnp.float32),
        ],
    )(hs, g, b, rw, rb)


def _ffn_body(te_ref, xs_ref, w1_ref, b1_ref, w2_ref, b2_ref, ws_ref, ys_ref):
    t = pl.program_id(0)

    @pl.when(t < te_ref[NT])
    def _():
        x = xs_ref[...]
        a = jnp.dot(x, w1_ref[0], preferred_element_type=jnp.float32)
        a = a + b1_ref[0]
        a = a * 0.5 * (1.0 + jax.lax.erf(a * 0.7071067811865476))
        y = jnp.dot(a, w2_ref[0], preferred_element_type=jnp.float32)
        y = y + b2_ref[0]
        ys_ref[...] = y * ws_ref[0]


def _ffn(xs, w1, b1, w2, b2, ws3, te_tab):
    grid_spec = pltpu.PrefetchScalarGridSpec(
        num_scalar_prefetch=1,
        grid=(NT,),
        in_specs=[
            pl.BlockSpec((T, H), lambda t, te: (jnp.minimum(t, te[NT] - 1), 0)),
            pl.BlockSpec((1, H, I), lambda t, te: (te[t], 0, 0)),
            pl.BlockSpec((1, 1, I), lambda t, te: (te[t], 0, 0)),
            pl.BlockSpec((1, I, H), lambda t, te: (te[t], 0, 0)),
            pl.BlockSpec((1, 1, H), lambda t, te: (te[t], 0, 0)),
            pl.BlockSpec((1, T, 1),
                         lambda t, te: (jnp.minimum(t, te[NT] - 1), 0, 0)),
        ],
        out_specs=pl.BlockSpec(
            (T, H), lambda t, te: (jnp.minimum(t, te[NT] - 1), 0)),
    )
    return pl.pallas_call(
        _ffn_body,
        grid_spec=grid_spec,
        out_shape=jax.ShapeDtypeStruct((P, H), jnp.float32),
    )(te_tab, xs, w1, b1, w2, b2, ws3)


_SC_MESH = plsc.VectorSubcoreMesh(core_axis_name="c", subcore_axis_name="s")
_NW = 32
_GCH = 32
_CCH = 32


_TPW = S // _NW


def _sc_dispatch_body(h_hbm, dest2_hbm, xs_hbm,
                      buf, idx_v, lsem, sem):
    wid = lax.axis_index("s") * 2 + lax.axis_index("c")
    t0 = wid * _TPW
    pltpu.sync_copy(dest2_hbm.at[0, pl.ds(t0, _TPW)], idx_v.at[0])
    pltpu.sync_copy(dest2_hbm.at[1, pl.ds(t0, _TPW)], idx_v.at[1])
    pltpu.async_copy(h_hbm.at[pl.ds(t0, _TPW)], buf, lsem).wait()
    cps = [
        pltpu.async_copy(buf, xs_hbm.at[idx_v.at[0]], sem),
        pltpu.async_copy(buf, xs_hbm.at[idx_v.at[1]], sem),
    ]
    for c in cps:
        c.wait()


@functools.partial(
    pl.kernel, mesh=_SC_MESH,
    out_type=jax.ShapeDtypeStruct((P, H), jnp.float32),
    scratch_types=[
        pltpu.VMEM((_TPW, H), jnp.float32),
        pltpu.VMEM((K, _TPW), jnp.int32),
        pltpu.SemaphoreType.DMA,
        pltpu.SemaphoreType.DMA,
    ],
)
def _sc_dispatch(h_hbm, dest2_hbm, xs_hbm, buf, idx_v, lsem, sem):
    _sc_dispatch_body(h_hbm, dest2_hbm, xs_hbm, buf, idx_v, lsem, sem)


def _sc_combine_body(ys_hbm, dest_hbm, res_hbm, out_hbm,
                     idx_v, rows_v, acc_v, sem):
    wid = lax.axis_index("s") * 2 + lax.axis_index("c")
    for c in range(_TPW // _CCH):
        t0 = wid * _TPW + c * _CCH
        pltpu.sync_copy(dest_hbm.at[pl.ds(K * t0, K * _CCH)], idx_v)
        gat = pltpu.async_copy(ys_hbm.at[idx_v], rows_v, sem)
        pltpu.sync_copy(res_hbm.at[pl.ds(t0, _CCH)], acc_v)
        gat.wait()

        def _row(i, _):
            def _vec(j, _):
                sl = pl.ds(j * 16, 16)
                acc_v[i, sl] = (acc_v[i, sl] + rows_v[2 * i, sl]
                                + rows_v[2 * i + 1, sl])
                return 0
            return lax.fori_loop(0, H // 16, _vec, 0)

        lax.fori_loop(0, _CCH, _row, 0)
        pltpu.sync_copy(acc_v, out_hbm.at[pl.ds(t0, _CCH)])


@functools.partial(
    pl.kernel, mesh=_SC_MESH,
    out_type=jax.ShapeDtypeStruct((S, H), jnp.float32),
    scratch_types=[
        pltpu.VMEM((K * _CCH,), jnp.int32),
        pltpu.VMEM((K * _CCH, H), jnp.float32),
        pltpu.VMEM((_CCH, H), jnp.float32),
        pltpu.SemaphoreType.DMA,
    ],
)
def _sc_combine(ys_hbm, dest_hbm, res_hbm, out_hbm,
                idx_v, rows_v, acc_v, sem):
    _sc_combine_body(ys_hbm, dest_hbm, res_hbm, out_hbm,
                     idx_v, rows_v, acc_v, sem)


def _combine_body(res_ref, yu_ref, out_ref):
    out_ref[...] = res_ref[...] + yu_ref[:, 0, :] + yu_ref[:, 1, :]


def _combine(res, yu):
    TC = 256
    return pl.pallas_call(
        _combine_body,
        grid=(S // TC,),
        in_specs=[
            pl.BlockSpec((TC, H), lambda i: (i, 0)),
            pl.BlockSpec((TC, K, H), lambda i: (i, 0, 0)),
        ],
        out_specs=pl.BlockSpec((TC, H), lambda i: (i, 0)),
        out_shape=jax.ShapeDtypeStruct((S, H), jnp.float32),
    )(res, yu)


def kernel(hidden_states, ln_gamma, ln_beta, router_w, router_b,
           w1, b1, w2, b2, training):
    hs = hidden_states.reshape(S, H)
    g2 = ln_gamma.reshape(1, H)
    b2d = ln_beta.reshape(1, H)
    rb2 = router_b.reshape(1, E)
    b1r = b1.reshape(E, 1, I)
    b2r = b2.reshape(E, 1, H)

    h, idx2, wts2, loss = _router(hs, g2, b2d, router_w, rb2)

    ef = idx2.reshape(-1)
    wf = wts2.reshape(-1)
    onehot = jax.nn.one_hot(ef, E, dtype=jnp.float32)
    csum = jnp.cumsum(onehot, axis=0)
    rank = jnp.sum(csum * onehot, axis=1).astype(jnp.int32) - 1
    counts = csum[-1].astype(jnp.int32)
    tiles_e = (counts + (T - 1)) // T
    tcum = jnp.cumsum(tiles_e)
    pstart = T * (tcum - tiles_e)
    dest = (jnp.sum(onehot * pstart.astype(jnp.float32)[None, :],
                    axis=1).astype(jnp.int32) + rank)
    dest2 = dest.reshape(S, K).T
    ntiles = tcum[-1]
    tile_id = jnp.arange(NT, dtype=jnp.int32)
    te_raw = jnp.searchsorted(tcum, tile_id, side='right').astype(jnp.int32)
    last_e = jnp.searchsorted(tcum, ntiles - 1, side='right').astype(jnp.int32)
    te = jnp.where(tile_id < ntiles, jnp.minimum(te_raw, E - 1), last_e)
    te_tab = jnp.concatenate([te, ntiles.astype(jnp.int32)[None]])

    xs = _sc_dispatch(h, dest2)
    ws = jnp.zeros((P,), jnp.float32).at[dest].set(wf)
    ws3 = ws.reshape(NT, T, 1)

    ys = _ffn(xs, w1, b1r, w2, b2r, ws3, te_tab)

    out = _sc_combine(ys, dest, hs)

    return (out.reshape(1, S, H), loss[0, 0])
